# Initial kernel scaffold; baseline (speedup 1.0000x reference)
import jax
import jax.numpy as jnp
from jax.experimental import pallas as pl


def kernel(x, pos, batch, W1, b1, g1, be1, W2, b2, g2, be2, W3, b3):
    raise NotImplementedError("write your pallas kernel here")


# speedup vs baseline: 3.0860x; 3.0860x over previous
import functools

import jax
import jax.numpy as jnp
from jax import lax
from jax.experimental import pallas as pl
from jax.experimental.pallas import tpu as pltpu
from jax.experimental.pallas import tpu_sc as plsc

_N = 10000
_S = 2500
_K = 64
_R2 = 0.2 * 0.2
_NPAD = 10240
_FC = 1280
_SPAD = 2504
_TILES = _SPAD // 8
_EDGES = _SPAD * _K
_EPAD = 163840
_BIGI = 2 ** 30


def _fps_body(p_ref, sel_ref):
    px = p_ref[0:8, :]
    py = p_ref[8:16, :]
    pz = p_ref[16:24, :]
    fi = (lax.broadcasted_iota(jnp.int32, (8, _FC), 0) * _FC
          + lax.broadcasted_iota(jnp.int32, (8, _FC), 1))
    live = fi < _N
    neg = jnp.float32(-jnp.inf)

    def coord(n):
        sel = fi == n
        return (jnp.sum(jnp.where(sel, px, 0.0)),
                jnp.sum(jnp.where(sel, py, 0.0)),
                jnp.sum(jnp.where(sel, pz, 0.0)))

    def dist(q):
        dx = px - q[0]
        dy = py - q[1]
        dz = pz - q[2]
        return dx * dx + dy * dy + dz * dz

    dists = jnp.where(live, dist(coord(jnp.int32(0))), neg)
    si = (lax.broadcasted_iota(jnp.int32, (8, 320), 0) * 320
          + lax.broadcasted_iota(jnp.int32, (8, 320), 1))
    acc = jnp.zeros((8, 320), jnp.int32)

    def body(i, carry):
        d, a = carry
        m = jnp.max(d)
        nxt = jnp.min(jnp.where(d == m, fi, _BIGI))
        a = jnp.where(si == i, nxt, a)
        d = jnp.minimum(d, dist(coord(nxt)))
        return d, a

    _, acc = lax.fori_loop(1, _S, body, (dists, acc))
    sel_ref[...] = acc


def _fps_call(p):
    return pl.pallas_call(
        _fps_body,
        out_shape=jax.ShapeDtypeStruct((8, 320), jnp.int32),
    )(p)


def _nbr_body(d2_ref, py_ref, w1p_ref, nbr_ref, val_ref, c_ref):
    ci = lax.broadcasted_iota(jnp.int32, (8, _NPAD), 1)
    inf = jnp.float32(jnp.inf)
    work = d2_ref[...]
    col = lax.broadcasted_iota(jnp.int32, (8, _K), 1)

    def body(t, carry):
        w, nbr, val = carry
        m = jnp.min(w, axis=1, keepdims=True)
        idx = jnp.min(jnp.where(w == m, ci, _BIGI), axis=1,
                      keepdims=True)
        nbr = jnp.where(col == t, idx, nbr)
        val = jnp.where(col == t, m, val)
        w = jnp.where(ci == idx, inf, w)
        return w, nbr, val

    nbr0 = jnp.zeros((8, _K), jnp.int32)
    val0 = jnp.full((8, _K), inf, jnp.float32)
    _, nbr, val = lax.fori_loop(0, _K, body, (work, nbr0, val0))
    row = (pl.program_id(0) * 8
           + lax.broadcasted_iota(jnp.int32, (8, _K), 0))
    good = (val <= jnp.float32(_R2)) & (row < _S)
    nbr_ref[...] = nbr
    val_ref[...] = good.astype(jnp.float32)
    c_ref[...] = jnp.dot(py_ref[...], w1p_ref[...],
                         preferred_element_type=jnp.float32)


def _nbr_call(d2p, py, w1p):
    return pl.pallas_call(
        _nbr_body,
        grid=(_TILES,),
        in_specs=[
            pl.BlockSpec((8, _NPAD), lambda i: (i, 0)),
            pl.BlockSpec((8, 8), lambda i: (i, 0)),
            pl.BlockSpec((8, 128), lambda i: (0, 0)),
        ],
        out_specs=[
            pl.BlockSpec((8, _K), lambda i: (i, 0)),
            pl.BlockSpec((8, _K), lambda i: (i, 0)),
            pl.BlockSpec((8, 128), lambda i: (i, 0)),
        ],
        out_shape=[
            jax.ShapeDtypeStruct((_SPAD, _K), jnp.int32),
            jax.ShapeDtypeStruct((_SPAD, _K), jnp.float32),
            jax.ShapeDtypeStruct((_SPAD, 128), jnp.float32),
        ],
    )(d2p, py, w1p)


def _g1_body(x_ref, p_ref, wx_ref, wp_ref, o_ref):
    o_ref[...] = (
        jnp.dot(x_ref[...], wx_ref[...], preferred_element_type=jnp.float32)
        + jnp.dot(p_ref[...], wp_ref[...], preferred_element_type=jnp.float32))


def _g1_call(xp, pp, w1x, w1p):
    return pl.pallas_call(
        _g1_body,
        grid=(_NPAD // 256,),
        in_specs=[
            pl.BlockSpec((256, 128), lambda i: (i, 0)),
            pl.BlockSpec((256, 8), lambda i: (i, 0)),
            pl.BlockSpec((128, 128), lambda i: (0, 0)),
            pl.BlockSpec((8, 128), lambda i: (0, 0)),
        ],
        out_specs=pl.BlockSpec((256, 128), lambda i: (i, 0)),
        out_shape=jax.ShapeDtypeStruct((_NPAD, 128), jnp.float32),
    )(xp, pp, w1x, w1p)


def _sc_gather(table, idx):
    mesh = plsc.VectorSubcoreMesh(core_axis_name="c", subcore_axis_name="s")

    @functools.partial(
        pl.kernel,
        mesh=mesh,
        out_type=jax.ShapeDtypeStruct((_EPAD, 128), jnp.float32),
        scratch_types=[
            pltpu.VMEM((128,), jnp.int32),
            pltpu.VMEM((128, 128), jnp.float32),
            pltpu.SemaphoreType.DMA,
        ],
    )
    def k(table_hbm, idx_hbm, out_hbm, idx_v, rows_v, sem):
        wid = lax.axis_index("s") * 2 + lax.axis_index("c")
        base = wid * (_EPAD // 32)
        for t in range(_EPAD // 32 // 128):
            off = base + t * 128
            pltpu.sync_copy(idx_hbm.at[pl.ds(off, 128)], idx_v)
            pltpu.async_copy(table_hbm.at[idx_v], rows_v, sem).wait()
            pltpu.sync_copy(rows_v, out_hbm.at[pl.ds(off, 128)])

    return k(table, idx)


def _statsA_body(e_ref, c_ref, v_ref, b_ref, ssum_ref, ssq_ref, cnt_ref):
    @pl.when(pl.program_id(0) == 0)
    def _():
        ssum_ref[...] = jnp.zeros_like(ssum_ref)
        ssq_ref[...] = jnp.zeros_like(ssq_ref)
        cnt_ref[...] = jnp.zeros_like(cnt_ref)

    z = (e_ref[...].reshape(8, _K, 128) - c_ref[...][:, None, :]
         + b_ref[...][None, :, :])
    w = v_ref[...][:, :, None]
    ssum_ref[...] += jnp.sum(z * w, axis=1)
    ssq_ref[...] += jnp.sum(z * z * w, axis=1)
    cnt_ref[...] += jnp.broadcast_to(
        jnp.sum(v_ref[...], axis=1, keepdims=True), (8, 128))


def _statsA_call(e1, c, valid, b1r):
    return pl.pallas_call(
        _statsA_body,
        grid=(_TILES,),
        in_specs=[
            pl.BlockSpec((8 * _K, 128), lambda i: (i, 0)),
            pl.BlockSpec((8, 128), lambda i: (i, 0)),
            pl.BlockSpec((8, _K), lambda i: (i, 0)),
            pl.BlockSpec((1, 128), lambda i: (0, 0)),
        ],
        out_specs=[
            pl.BlockSpec((8, 128), lambda i: (0, 0)),
            pl.BlockSpec((8, 128), lambda i: (0, 0)),
            pl.BlockSpec((8, 128), lambda i: (0, 0)),
        ],
        out_shape=[
            jax.ShapeDtypeStruct((8, 128), jnp.float32),
            jax.ShapeDtypeStruct((8, 128), jnp.float32),
            jax.ShapeDtypeStruct((8, 128), jnp.float32),
        ],
    )(e1, c, valid, b1r)


def _passB_body(e_ref, c_ref, v_ref, b_ref, sc_ref, sh_ref, w2_ref, b2_ref,
                z2_ref, ssum_ref, ssq_ref):
    @pl.when(pl.program_id(0) == 0)
    def _():
        ssum_ref[...] = jnp.zeros_like(ssum_ref)
        ssq_ref[...] = jnp.zeros_like(ssq_ref)

    z1 = (e_ref[...].reshape(8, _K, 128) - c_ref[...][:, None, :]
          + b_ref[...][None, :, :])
    h1 = jnp.maximum(z1 * sc_ref[...][None, :, :] + sh_ref[...][None, :, :],
                     0.0)
    z2 = (jnp.dot(h1.reshape(8 * _K, 128), w2_ref[...],
                  preferred_element_type=jnp.float32) + b2_ref[...])
    z2_ref[...] = z2
    w = v_ref[...][:, :, None]
    z2r = z2.reshape(8, _K, 128)
    ssum_ref[...] += jnp.sum(z2r * w, axis=1)
    ssq_ref[...] += jnp.sum(z2r * z2r * w, axis=1)


def _passB_call(e1, c, valid, b1r, sc1, sh1, w2, b2r):
    return pl.pallas_call(
        _passB_body,
        grid=(_TILES,),
        in_specs=[
            pl.BlockSpec((8 * _K, 128), lambda i: (i, 0)),
            pl.BlockSpec((8, 128), lambda i: (i, 0)),
            pl.BlockSpec((8, _K), lambda i: (i, 0)),
            pl.BlockSpec((1, 128), lambda i: (0, 0)),
            pl.BlockSpec((1, 128), lambda i: (0, 0)),
            pl.BlockSpec((1, 128), lambda i: (0, 0)),
            pl.BlockSpec((128, 128), lambda i: (0, 0)),
            pl.BlockSpec((1, 128), lambda i: (0, 0)),
        ],
        out_specs=[
            pl.BlockSpec((8 * _K, 128), lambda i: (i, 0)),
            pl.BlockSpec((8, 128), lambda i: (0, 0)),
            pl.BlockSpec((8, 128), lambda i: (0, 0)),
        ],
        out_shape=[
            jax.ShapeDtypeStruct((_EDGES, 128), jnp.float32),
            jax.ShapeDtypeStruct((8, 128), jnp.float32),
            jax.ShapeDtypeStruct((8, 128), jnp.float32),
        ],
    )(e1, c, valid, b1r, sc1, sh1, w2, b2r)


def _passC_body(z2_ref, v_ref, sc_ref, sh_ref, w3_ref, b3_ref, o_ref):
    h2 = jnp.maximum(z2_ref[...] * sc_ref[...] + sh_ref[...], 0.0)
    z3 = (jnp.dot(h2, w3_ref[...], preferred_element_type=jnp.float32)
          + b3_ref[...])
    z3r = z3.reshape(8, _K, 128)
    msk = v_ref[...][:, :, None] > 0.0
    m = jnp.max(jnp.where(msk, z3r, jnp.float32(-jnp.inf)), axis=1)
    o_ref[...] = jnp.where(jnp.isfinite(m), m, 0.0)


def _passC_call(z2, valid, sc2, sh2, w3, b3r):
    return pl.pallas_call(
        _passC_body,
        grid=(_TILES,),
        in_specs=[
            pl.BlockSpec((8 * _K, 128), lambda i: (i, 0)),
            pl.BlockSpec((8, _K), lambda i: (i, 0)),
            pl.BlockSpec((1, 128), lambda i: (0, 0)),
            pl.BlockSpec((1, 128), lambda i: (0, 0)),
            pl.BlockSpec((128, 128), lambda i: (0, 0)),
            pl.BlockSpec((1, 128), lambda i: (0, 0)),
        ],
        out_specs=pl.BlockSpec((8, 128), lambda i: (i, 0)),
        out_shape=jax.ShapeDtypeStruct((_SPAD, 128), jnp.float32),
    )(z2, valid, sc2, sh2, w3, b3r)


def _bn_coeffs(ssum, ssq, cnt, gamma, beta):
    s = jnp.sum(ssum, axis=0)
    q = jnp.sum(ssq, axis=0)
    mean = s / cnt
    var = jnp.maximum(q / cnt - mean * mean, 0.0)
    scale = gamma / jnp.sqrt(var + 1e-5)
    shift = beta - mean * scale
    return scale.reshape(1, 128), shift.reshape(1, 128)


def kernel(x, pos, batch, W1, b1, g1, be1, W2, b2, g2, be2, W3, b3):
    f32 = jnp.float32
    x = x.astype(f32)
    pos = pos.astype(f32)

    pt = jnp.transpose(pos)
    ptp = jnp.pad(pt, ((0, 0), (0, _NPAD - _N)))
    p_in = jnp.concatenate([ptp[0].reshape(8, _FC),
                            ptp[1].reshape(8, _FC),
                            ptp[2].reshape(8, _FC)], axis=0)
    sel = _fps_call(p_in).reshape(-1)[:_S]
    pos_y = jnp.take(pos, sel, axis=0)
    batch_y = jnp.take(batch, sel)

    yn = jnp.sum(pos_y ** 2, axis=1)[:, None]
    xn = jnp.sum(pos ** 2, axis=1)[None, :]
    d2 = yn + xn - 2.0 * (pos_y @ pos.T)
    d2 = jnp.where(batch[None, :] == batch_y[:, None], d2, jnp.inf)
    d2p = jnp.pad(d2, ((0, _SPAD - _S), (0, _NPAD - _N)),
                  constant_values=jnp.inf)
    py = jnp.pad(pos_y, ((0, _SPAD - _S), (0, 5)))
    w1p = jnp.pad(W1[128:], ((0, 5), (0, 0)))
    nbr, valid, c = _nbr_call(d2p, py, w1p)

    xp = jnp.pad(x, ((0, _NPAD - _N), (0, 0)))
    pp = jnp.pad(pos, ((0, _NPAD - _N), (0, 5)))
    g1t = _g1_call(xp, pp, W1[:128], w1p)
    idx = jnp.pad(nbr.reshape(-1), (0, _EPAD - _EDGES))
    e1 = _sc_gather(g1t, idx)

    b1r = b1.reshape(1, 128)
    ssum1, ssq1, cnt1 = _statsA_call(e1, c, valid, b1r)
    cnt = jnp.maximum(jnp.sum(cnt1[:, 0]), 1.0)
    sc1, sh1 = _bn_coeffs(ssum1, ssq1, cnt, g1, be1)
    z2, ssum2, ssq2 = _passB_call(e1, c, valid, b1r, sc1, sh1, W2,
                                  b2.reshape(1, 128))
    sc2, sh2 = _bn_coeffs(ssum2, ssq2, cnt, g2, be2)
    outp = _passC_call(z2, valid, sc2, sh2, W3, b3.reshape(1, 128))
    return (outp[:_S], pos_y, batch_y)


# speedup vs baseline: 5.5625x; 1.8025x over previous
import functools

import jax
import jax.numpy as jnp
from jax import lax
from jax.experimental import pallas as pl
from jax.experimental.pallas import tpu as pltpu
from jax.experimental.pallas import tpu_sc as plsc

_N = 10000
_S = 2500
_K = 64
_R2 = 0.2 * 0.2
_NPAD = 10240
_FC = 1280
_SPAD = 2560
_TILES = _SPAD // 8
_NROW = 64
_NT = _SPAD // _NROW
_EDGES = _SPAD * _K
_EPAD = _EDGES
_BIGI = 2 ** 30


def _fps_body(p_ref, sel_ref):
    px = p_ref[0:8, :]
    py = p_ref[8:16, :]
    pz = p_ref[16:24, :]
    fi = (lax.broadcasted_iota(jnp.int32, (8, _FC), 0) * _FC
          + lax.broadcasted_iota(jnp.int32, (8, _FC), 1))
    live = fi < _N
    neg = jnp.float32(-jnp.inf)

    def coord(n):
        sel = fi == n
        return (jnp.sum(jnp.where(sel, px, 0.0)),
                jnp.sum(jnp.where(sel, py, 0.0)),
                jnp.sum(jnp.where(sel, pz, 0.0)))

    def dist(q):
        dx = px - q[0]
        dy = py - q[1]
        dz = pz - q[2]
        return dx * dx + dy * dy + dz * dz

    dists = jnp.where(live, dist(coord(jnp.int32(0))), neg)
    si = (lax.broadcasted_iota(jnp.int32, (8, 320), 0) * 320
          + lax.broadcasted_iota(jnp.int32, (8, 320), 1))
    acc = jnp.zeros((8, 320), jnp.int32)

    def body(i, carry):
        d, a = carry
        m = jnp.max(d)
        nxt = jnp.min(jnp.where(d == m, fi, _BIGI))
        a = jnp.where(si == i, nxt, a)
        d = jnp.minimum(d, dist(coord(nxt)))
        return d, a

    _, acc = lax.fori_loop(1, _S, body, (dists, acc))
    sel_ref[...] = acc


def _fps_call(p):
    return pl.pallas_call(
        _fps_body,
        out_shape=jax.ShapeDtypeStruct((8, 320), jnp.int32),
    )(p)


def _nbr_body(d2_ref, py_ref, w1p_ref, nbr_ref, val_ref, c_ref):
    ci = lax.broadcasted_iota(jnp.int32, (_NROW, _NPAD), 1)
    inf = jnp.float32(jnp.inf)
    work = d2_ref[...]
    col = lax.broadcasted_iota(jnp.int32, (_NROW, _K), 1)

    def body(t, carry):
        w, nbr, val = carry
        m = jnp.min(w, axis=1, keepdims=True)
        idx = jnp.min(jnp.where(w == m, ci, _BIGI), axis=1,
                      keepdims=True)
        nbr = jnp.where(col == t, idx, nbr)
        val = jnp.where(col == t, m, val)
        w = jnp.where(ci == idx, inf, w)
        return w, nbr, val

    nbr0 = jnp.zeros((_NROW, _K), jnp.int32)
    val0 = jnp.full((_NROW, _K), inf, jnp.float32)
    _, nbr, val = lax.fori_loop(0, _K, body, (work, nbr0, val0))
    row = (pl.program_id(0) * _NROW
           + lax.broadcasted_iota(jnp.int32, (_NROW, _K), 0))
    good = (val <= jnp.float32(_R2)) & (row < _S)
    nbr_ref[...] = nbr
    val_ref[...] = good.astype(jnp.float32)
    c_ref[...] = jnp.dot(py_ref[...], w1p_ref[...],
                         preferred_element_type=jnp.float32)


def _nbr_call(d2p, py, w1p):
    return pl.pallas_call(
        _nbr_body,
        grid=(_NT,),
        in_specs=[
            pl.BlockSpec((_NROW, _NPAD), lambda i: (i, 0)),
            pl.BlockSpec((_NROW, 8), lambda i: (i, 0)),
            pl.BlockSpec((8, 128), lambda i: (0, 0)),
        ],
        out_specs=[
            pl.BlockSpec((_NROW, _K), lambda i: (i, 0)),
            pl.BlockSpec((_NROW, _K), lambda i: (i, 0)),
            pl.BlockSpec((_NROW, 128), lambda i: (i, 0)),
        ],
        out_shape=[
            jax.ShapeDtypeStruct((_SPAD, _K), jnp.int32),
            jax.ShapeDtypeStruct((_SPAD, _K), jnp.float32),
            jax.ShapeDtypeStruct((_SPAD, 128), jnp.float32),
        ],
    )(d2p, py, w1p)


def _g1_body(x_ref, p_ref, wx_ref, wp_ref, o_ref):
    o_ref[...] = (
        jnp.dot(x_ref[...], wx_ref[...], preferred_element_type=jnp.float32)
        + jnp.dot(p_ref[...], wp_ref[...], preferred_element_type=jnp.float32))


def _g1_call(xp, pp, w1x, w1p):
    return pl.pallas_call(
        _g1_body,
        grid=(_NPAD // 256,),
        in_specs=[
            pl.BlockSpec((256, 128), lambda i: (i, 0)),
            pl.BlockSpec((256, 8), lambda i: (i, 0)),
            pl.BlockSpec((128, 128), lambda i: (0, 0)),
            pl.BlockSpec((8, 128), lambda i: (0, 0)),
        ],
        out_specs=pl.BlockSpec((256, 128), lambda i: (i, 0)),
        out_shape=jax.ShapeDtypeStruct((_NPAD, 128), jnp.float32),
    )(xp, pp, w1x, w1p)


def _sc_gather(table, idx):
    mesh = plsc.VectorSubcoreMesh(core_axis_name="c", subcore_axis_name="s")

    @functools.partial(
        pl.kernel,
        mesh=mesh,
        out_type=jax.ShapeDtypeStruct((_EPAD, 128), jnp.float32),
        scratch_types=[
            pltpu.VMEM((128,), jnp.int32),
            pltpu.VMEM((128, 128), jnp.float32),
            pltpu.SemaphoreType.DMA,
        ],
    )
    def k(table_hbm, idx_hbm, out_hbm, idx_v, rows_v, sem):
        wid = lax.axis_index("s") * 2 + lax.axis_index("c")
        base = wid * (_EPAD // 32)
        for t in range(_EPAD // 32 // 128):
            off = base + t * 128
            pltpu.sync_copy(idx_hbm.at[pl.ds(off, 128)], idx_v)
            pltpu.async_copy(table_hbm.at[idx_v], rows_v, sem).wait()
            pltpu.sync_copy(rows_v, out_hbm.at[pl.ds(off, 128)])

    return k(table, idx)


def _statsA_body(e_ref, c_ref, v_ref, b_ref, ssum_ref, ssq_ref, cnt_ref):
    @pl.when(pl.program_id(0) == 0)
    def _():
        ssum_ref[...] = jnp.zeros_like(ssum_ref)
        ssq_ref[...] = jnp.zeros_like(ssq_ref)
        cnt_ref[...] = jnp.zeros_like(cnt_ref)

    z = (e_ref[...].reshape(8, _K, 128) - c_ref[...][:, None, :]
         + b_ref[...][None, :, :])
    w = v_ref[...][:, :, None]
    ssum_ref[...] += jnp.sum(z * w, axis=1)
    ssq_ref[...] += jnp.sum(z * z * w, axis=1)
    cnt_ref[...] += jnp.broadcast_to(
        jnp.sum(v_ref[...], axis=1, keepdims=True), (8, 128))


def _statsA_call(e1, c, valid, b1r):
    return pl.pallas_call(
        _statsA_body,
        grid=(_TILES,),
        in_specs=[
            pl.BlockSpec((8 * _K, 128), lambda i: (i, 0)),
            pl.BlockSpec((8, 128), lambda i: (i, 0)),
            pl.BlockSpec((8, _K), lambda i: (i, 0)),
            pl.BlockSpec((1, 128), lambda i: (0, 0)),
        ],
        out_specs=[
            pl.BlockSpec((8, 128), lambda i: (0, 0)),
            pl.BlockSpec((8, 128), lambda i: (0, 0)),
            pl.BlockSpec((8, 128), lambda i: (0, 0)),
        ],
        out_shape=[
            jax.ShapeDtypeStruct((8, 128), jnp.float32),
            jax.ShapeDtypeStruct((8, 128), jnp.float32),
            jax.ShapeDtypeStruct((8, 128), jnp.float32),
        ],
    )(e1, c, valid, b1r)


def _passB_body(e_ref, c_ref, v_ref, b_ref, sc_ref, sh_ref, w2_ref, b2_ref,
                z2_ref, ssum_ref, ssq_ref):
    @pl.when(pl.program_id(0) == 0)
    def _():
        ssum_ref[...] = jnp.zeros_like(ssum_ref)
        ssq_ref[...] = jnp.zeros_like(ssq_ref)

    z1 = (e_ref[...].reshape(8, _K, 128) - c_ref[...][:, None, :]
          + b_ref[...][None, :, :])
    h1 = jnp.maximum(z1 * sc_ref[...][None, :, :] + sh_ref[...][None, :, :],
                     0.0)
    z2 = (jnp.dot(h1.reshape(8 * _K, 128), w2_ref[...],
                  preferred_element_type=jnp.float32) + b2_ref[...])
    z2_ref[...] = z2
    w = v_ref[...][:, :, None]
    z2r = z2.reshape(8, _K, 128)
    ssum_ref[...] += jnp.sum(z2r * w, axis=1)
    ssq_ref[...] += jnp.sum(z2r * z2r * w, axis=1)


def _passB_call(e1, c, valid, b1r, sc1, sh1, w2, b2r):
    return pl.pallas_call(
        _passB_body,
        grid=(_TILES,),
        in_specs=[
            pl.BlockSpec((8 * _K, 128), lambda i: (i, 0)),
            pl.BlockSpec((8, 128), lambda i: (i, 0)),
            pl.BlockSpec((8, _K), lambda i: (i, 0)),
            pl.BlockSpec((1, 128), lambda i: (0, 0)),
            pl.BlockSpec((1, 128), lambda i: (0, 0)),
            pl.BlockSpec((1, 128), lambda i: (0, 0)),
            pl.BlockSpec((128, 128), lambda i: (0, 0)),
            pl.BlockSpec((1, 128), lambda i: (0, 0)),
        ],
        out_specs=[
            pl.BlockSpec((8 * _K, 128), lambda i: (i, 0)),
            pl.BlockSpec((8, 128), lambda i: (0, 0)),
            pl.BlockSpec((8, 128), lambda i: (0, 0)),
        ],
        out_shape=[
            jax.ShapeDtypeStruct((_EDGES, 128), jnp.float32),
            jax.ShapeDtypeStruct((8, 128), jnp.float32),
            jax.ShapeDtypeStruct((8, 128), jnp.float32),
        ],
    )(e1, c, valid, b1r, sc1, sh1, w2, b2r)


def _passC_body(z2_ref, v_ref, sc_ref, sh_ref, w3_ref, b3_ref, o_ref):
    h2 = jnp.maximum(z2_ref[...] * sc_ref[...] + sh_ref[...], 0.0)
    z3 = (jnp.dot(h2, w3_ref[...], preferred_element_type=jnp.float32)
          + b3_ref[...])
    z3r = z3.reshape(8, _K, 128)
    msk = v_ref[...][:, :, None] > 0.0
    m = jnp.max(jnp.where(msk, z3r, jnp.float32(-jnp.inf)), axis=1)
    o_ref[...] = jnp.where(jnp.isfinite(m), m, 0.0)


def _passC_call(z2, valid, sc2, sh2, w3, b3r):
    return pl.pallas_call(
        _passC_body,
        grid=(_TILES,),
        in_specs=[
            pl.BlockSpec((8 * _K, 128), lambda i: (i, 0)),
            pl.BlockSpec((8, _K), lambda i: (i, 0)),
            pl.BlockSpec((1, 128), lambda i: (0, 0)),
            pl.BlockSpec((1, 128), lambda i: (0, 0)),
            pl.BlockSpec((128, 128), lambda i: (0, 0)),
            pl.BlockSpec((1, 128), lambda i: (0, 0)),
        ],
        out_specs=pl.BlockSpec((8, 128), lambda i: (i, 0)),
        out_shape=jax.ShapeDtypeStruct((_SPAD, 128), jnp.float32),
    )(z2, valid, sc2, sh2, w3, b3r)


def _bn_coeffs(ssum, ssq, cnt, gamma, beta):
    s = jnp.sum(ssum, axis=0)
    q = jnp.sum(ssq, axis=0)
    mean = s / cnt
    var = jnp.maximum(q / cnt - mean * mean, 0.0)
    scale = gamma / jnp.sqrt(var + 1e-5)
    shift = beta - mean * scale
    return scale.reshape(1, 128), shift.reshape(1, 128)


def kernel(x, pos, batch, W1, b1, g1, be1, W2, b2, g2, be2, W3, b3):
    f32 = jnp.float32
    x = x.astype(f32)
    pos = pos.astype(f32)

    pt = jnp.transpose(pos)
    ptp = jnp.pad(pt, ((0, 0), (0, _NPAD - _N)))
    p_in = jnp.concatenate([ptp[0].reshape(8, _FC),
                            ptp[1].reshape(8, _FC),
                            ptp[2].reshape(8, _FC)], axis=0)
    sel = _fps_call(p_in).reshape(-1)[:_S]
    pos_y = jnp.take(pos, sel, axis=0)
    batch_y = jnp.take(batch, sel)

    yn = jnp.sum(pos_y ** 2, axis=1)[:, None]
    xn = jnp.sum(pos ** 2, axis=1)[None, :]
    d2 = yn + xn - 2.0 * (pos_y @ pos.T)
    d2 = jnp.where(batch[None, :] == batch_y[:, None], d2, jnp.inf)
    d2p = jnp.pad(d2, ((0, _SPAD - _S), (0, _NPAD - _N)),
                  constant_values=jnp.inf)
    py = jnp.pad(pos_y, ((0, _SPAD - _S), (0, 5)))
    w1p = jnp.pad(W1[128:], ((0, 5), (0, 0)))
    nbr, valid, c = _nbr_call(d2p, py, w1p)

    xp = jnp.pad(x, ((0, _NPAD - _N), (0, 0)))
    pp = jnp.pad(pos, ((0, _NPAD - _N), (0, 5)))
    g1t = _g1_call(xp, pp, W1[:128], w1p)
    idx = nbr.reshape(-1)
    e1 = _sc_gather(g1t, idx)

    b1r = b1.reshape(1, 128)
    ssum1, ssq1, cnt1 = _statsA_call(e1, c, valid, b1r)
    cnt = jnp.maximum(jnp.sum(cnt1[:, 0]), 1.0)
    sc1, sh1 = _bn_coeffs(ssum1, ssq1, cnt, g1, be1)
    z2, ssum2, ssq2 = _passB_call(e1, c, valid, b1r, sc1, sh1, W2,
                                  b2.reshape(1, 128))
    sc2, sh2 = _bn_coeffs(ssum2, ssq2, cnt, g2, be2)
    outp = _passC_call(z2, valid, sc2, sh2, W3, b3.reshape(1, 128))
    return (outp[:_S], pos_y, batch_y)


# speedup vs baseline: 5.8135x; 1.0451x over previous
import functools

import jax
import jax.numpy as jnp
from jax import lax
from jax.experimental import pallas as pl
from jax.experimental.pallas import tpu as pltpu
from jax.experimental.pallas import tpu_sc as plsc

_N = 10000
_S = 2500
_K = 64
_R2 = 0.2 * 0.2
_NPAD = 10240
_FC = 1280
_SPAD = 2560
_TILES = _SPAD // 8
_NROW = 128
_NT = _SPAD // _NROW
_EDGES = _SPAD * _K
_EPAD = _EDGES
_BIGI = 2 ** 30


def _fps_body(p_ref, sel_ref):
    px = p_ref[0:8, :]
    py = p_ref[8:16, :]
    pz = p_ref[16:24, :]
    fi = (lax.broadcasted_iota(jnp.int32, (8, _FC), 0) * _FC
          + lax.broadcasted_iota(jnp.int32, (8, _FC), 1))
    live = fi < _N
    neg = jnp.float32(-jnp.inf)

    def coord(n):
        sel = fi == n
        return (jnp.sum(jnp.where(sel, px, 0.0)),
                jnp.sum(jnp.where(sel, py, 0.0)),
                jnp.sum(jnp.where(sel, pz, 0.0)))

    def dist(q):
        dx = px - q[0]
        dy = py - q[1]
        dz = pz - q[2]
        return dx * dx + dy * dy + dz * dz

    dists = jnp.where(live, dist(coord(jnp.int32(0))), neg)
    si = (lax.broadcasted_iota(jnp.int32, (8, 320), 0) * 320
          + lax.broadcasted_iota(jnp.int32, (8, 320), 1))
    acc = jnp.zeros((8, 320), jnp.int32)

    def body(i, carry):
        d, a = carry
        m = jnp.max(d)
        nxt = jnp.min(jnp.where(d == m, fi, _BIGI))
        a = jnp.where(si == i, nxt, a)
        d = jnp.minimum(d, dist(coord(nxt)))
        return d, a

    _, acc = lax.fori_loop(1, _S, body, (dists, acc))
    sel_ref[...] = acc


def _fps_call(p):
    return pl.pallas_call(
        _fps_body,
        out_shape=jax.ShapeDtypeStruct((8, 320), jnp.int32),
    )(p)


def _nbr_body(d2_ref, py_ref, w1p_ref, nbr_ref, val_ref, c_ref):
    ci = lax.broadcasted_iota(jnp.int32, (_NROW, _NPAD), 1)
    inf = jnp.float32(jnp.inf)
    work = d2_ref[...]
    col = lax.broadcasted_iota(jnp.int32, (_NROW, _K), 1)

    def body(t, carry):
        w, nbr, val = carry
        m = jnp.min(w, axis=1, keepdims=True)
        idx = jnp.min(jnp.where(w == m, ci, _BIGI), axis=1,
                      keepdims=True)
        nbr = jnp.where(col == t, idx, nbr)
        val = jnp.where(col == t, m, val)
        w = jnp.where(ci == idx, inf, w)
        return w, nbr, val

    nbr0 = jnp.zeros((_NROW, _K), jnp.int32)
    val0 = jnp.full((_NROW, _K), inf, jnp.float32)
    _, nbr, val = lax.fori_loop(0, _K, body, (work, nbr0, val0))
    row = (pl.program_id(0) * _NROW
           + lax.broadcasted_iota(jnp.int32, (_NROW, _K), 0))
    good = (val <= jnp.float32(_R2)) & (row < _S)
    nbr_ref[...] = nbr
    val_ref[...] = good.astype(jnp.float32)
    c_ref[...] = jnp.dot(py_ref[...], w1p_ref[...],
                         preferred_element_type=jnp.float32)


def _nbr_call(d2p, py, w1p):
    return pl.pallas_call(
        _nbr_body,
        grid=(_NT,),
        in_specs=[
            pl.BlockSpec((_NROW, _NPAD), lambda i: (i, 0)),
            pl.BlockSpec((_NROW, 8), lambda i: (i, 0)),
            pl.BlockSpec((8, 128), lambda i: (0, 0)),
        ],
        out_specs=[
            pl.BlockSpec((_NROW, _K), lambda i: (i, 0)),
            pl.BlockSpec((_NROW, _K), lambda i: (i, 0)),
            pl.BlockSpec((_NROW, 128), lambda i: (i, 0)),
        ],
        out_shape=[
            jax.ShapeDtypeStruct((_SPAD, _K), jnp.int32),
            jax.ShapeDtypeStruct((_SPAD, _K), jnp.float32),
            jax.ShapeDtypeStruct((_SPAD, 128), jnp.float32),
        ],
    )(d2p, py, w1p)


def _g1_body(x_ref, p_ref, wx_ref, wp_ref, o_ref):
    o_ref[...] = (
        jnp.dot(x_ref[...], wx_ref[...], preferred_element_type=jnp.float32)
        + jnp.dot(p_ref[...], wp_ref[...], preferred_element_type=jnp.float32))


def _g1_call(xp, pp, w1x, w1p):
    return pl.pallas_call(
        _g1_body,
        grid=(_NPAD // 256,),
        in_specs=[
            pl.BlockSpec((256, 128), lambda i: (i, 0)),
            pl.BlockSpec((256, 8), lambda i: (i, 0)),
            pl.BlockSpec((128, 128), lambda i: (0, 0)),
            pl.BlockSpec((8, 128), lambda i: (0, 0)),
        ],
        out_specs=pl.BlockSpec((256, 128), lambda i: (i, 0)),
        out_shape=jax.ShapeDtypeStruct((_NPAD, 128), jnp.float32),
    )(xp, pp, w1x, w1p)


def _sc_gather(table, idx):
    mesh = plsc.VectorSubcoreMesh(core_axis_name="c", subcore_axis_name="s")

    @functools.partial(
        pl.kernel,
        mesh=mesh,
        out_type=jax.ShapeDtypeStruct((_EPAD, 128), jnp.float32),
        scratch_types=[
            pltpu.VMEM((128,), jnp.int32),
            pltpu.VMEM((128, 128), jnp.float32),
            pltpu.SemaphoreType.DMA,
        ],
    )
    def k(table_hbm, idx_hbm, out_hbm, idx_v, rows_v, sem):
        wid = lax.axis_index("s") * 2 + lax.axis_index("c")
        base = wid * (_EPAD // 32)
        for t in range(_EPAD // 32 // 128):
            off = base + t * 128
            pltpu.sync_copy(idx_hbm.at[pl.ds(off, 128)], idx_v)
            pltpu.async_copy(table_hbm.at[idx_v], rows_v, sem).wait()
            pltpu.sync_copy(rows_v, out_hbm.at[pl.ds(off, 128)])

    return k(table, idx)


def _statsA_body(e_ref, c_ref, v_ref, b_ref, ssum_ref, ssq_ref, cnt_ref):
    @pl.when(pl.program_id(0) == 0)
    def _():
        ssum_ref[...] = jnp.zeros_like(ssum_ref)
        ssq_ref[...] = jnp.zeros_like(ssq_ref)
        cnt_ref[...] = jnp.zeros_like(cnt_ref)

    z = (e_ref[...].reshape(8, _K, 128) - c_ref[...][:, None, :]
         + b_ref[...][None, :, :])
    w = v_ref[...][:, :, None]
    ssum_ref[...] += jnp.sum(z * w, axis=1)
    ssq_ref[...] += jnp.sum(z * z * w, axis=1)
    cnt_ref[...] += jnp.broadcast_to(
        jnp.sum(v_ref[...], axis=1, keepdims=True), (8, 128))


def _statsA_call(e1, c, valid, b1r):
    return pl.pallas_call(
        _statsA_body,
        grid=(_TILES,),
        in_specs=[
            pl.BlockSpec((8 * _K, 128), lambda i: (i, 0)),
            pl.BlockSpec((8, 128), lambda i: (i, 0)),
            pl.BlockSpec((8, _K), lambda i: (i, 0)),
            pl.BlockSpec((1, 128), lambda i: (0, 0)),
        ],
        out_specs=[
            pl.BlockSpec((8, 128), lambda i: (0, 0)),
            pl.BlockSpec((8, 128), lambda i: (0, 0)),
            pl.BlockSpec((8, 128), lambda i: (0, 0)),
        ],
        out_shape=[
            jax.ShapeDtypeStruct((8, 128), jnp.float32),
            jax.ShapeDtypeStruct((8, 128), jnp.float32),
            jax.ShapeDtypeStruct((8, 128), jnp.float32),
        ],
    )(e1, c, valid, b1r)


def _passB_body(e_ref, c_ref, v_ref, b_ref, sc_ref, sh_ref, w2_ref, b2_ref,
                z2_ref, ssum_ref, ssq_ref):
    @pl.when(pl.program_id(0) == 0)
    def _():
        ssum_ref[...] = jnp.zeros_like(ssum_ref)
        ssq_ref[...] = jnp.zeros_like(ssq_ref)

    z1 = (e_ref[...].reshape(8, _K, 128) - c_ref[...][:, None, :]
          + b_ref[...][None, :, :])
    h1 = jnp.maximum(z1 * sc_ref[...][None, :, :] + sh_ref[...][None, :, :],
                     0.0)
    z2 = (jnp.dot(h1.reshape(8 * _K, 128), w2_ref[...],
                  preferred_element_type=jnp.float32) + b2_ref[...])
    z2_ref[...] = z2
    w = v_ref[...][:, :, None]
    z2r = z2.reshape(8, _K, 128)
    ssum_ref[...] += jnp.sum(z2r * w, axis=1)
    ssq_ref[...] += jnp.sum(z2r * z2r * w, axis=1)


def _passB_call(e1, c, valid, b1r, sc1, sh1, w2, b2r):
    return pl.pallas_call(
        _passB_body,
        grid=(_TILES,),
        in_specs=[
            pl.BlockSpec((8 * _K, 128), lambda i: (i, 0)),
            pl.BlockSpec((8, 128), lambda i: (i, 0)),
            pl.BlockSpec((8, _K), lambda i: (i, 0)),
            pl.BlockSpec((1, 128), lambda i: (0, 0)),
            pl.BlockSpec((1, 128), lambda i: (0, 0)),
            pl.BlockSpec((1, 128), lambda i: (0, 0)),
            pl.BlockSpec((128, 128), lambda i: (0, 0)),
            pl.BlockSpec((1, 128), lambda i: (0, 0)),
        ],
        out_specs=[
            pl.BlockSpec((8 * _K, 128), lambda i: (i, 0)),
            pl.BlockSpec((8, 128), lambda i: (0, 0)),
            pl.BlockSpec((8, 128), lambda i: (0, 0)),
        ],
        out_shape=[
            jax.ShapeDtypeStruct((_EDGES, 128), jnp.float32),
            jax.ShapeDtypeStruct((8, 128), jnp.float32),
            jax.ShapeDtypeStruct((8, 128), jnp.float32),
        ],
    )(e1, c, valid, b1r, sc1, sh1, w2, b2r)


def _passC_body(z2_ref, v_ref, sc_ref, sh_ref, w3_ref, b3_ref, o_ref):
    h2 = jnp.maximum(z2_ref[...] * sc_ref[...] + sh_ref[...], 0.0)
    z3 = (jnp.dot(h2, w3_ref[...], preferred_element_type=jnp.float32)
          + b3_ref[...])
    z3r = z3.reshape(8, _K, 128)
    msk = v_ref[...][:, :, None] > 0.0
    m = jnp.max(jnp.where(msk, z3r, jnp.float32(-jnp.inf)), axis=1)
    o_ref[...] = jnp.where(jnp.isfinite(m), m, 0.0)


def _passC_call(z2, valid, sc2, sh2, w3, b3r):
    return pl.pallas_call(
        _passC_body,
        grid=(_TILES,),
        in_specs=[
            pl.BlockSpec((8 * _K, 128), lambda i: (i, 0)),
            pl.BlockSpec((8, _K), lambda i: (i, 0)),
            pl.BlockSpec((1, 128), lambda i: (0, 0)),
            pl.BlockSpec((1, 128), lambda i: (0, 0)),
            pl.BlockSpec((128, 128), lambda i: (0, 0)),
            pl.BlockSpec((1, 128), lambda i: (0, 0)),
        ],
        out_specs=pl.BlockSpec((8, 128), lambda i: (i, 0)),
        out_shape=jax.ShapeDtypeStruct((_SPAD, 128), jnp.float32),
    )(z2, valid, sc2, sh2, w3, b3r)


def _bn_coeffs(ssum, ssq, cnt, gamma, beta):
    s = jnp.sum(ssum, axis=0)
    q = jnp.sum(ssq, axis=0)
    mean = s / cnt
    var = jnp.maximum(q / cnt - mean * mean, 0.0)
    scale = gamma / jnp.sqrt(var + 1e-5)
    shift = beta - mean * scale
    return scale.reshape(1, 128), shift.reshape(1, 128)


def kernel(x, pos, batch, W1, b1, g1, be1, W2, b2, g2, be2, W3, b3):
    f32 = jnp.float32
    x = x.astype(f32)
    pos = pos.astype(f32)

    pt = jnp.transpose(pos)
    ptp = jnp.pad(pt, ((0, 0), (0, _NPAD - _N)))
    p_in = jnp.concatenate([ptp[0].reshape(8, _FC),
                            ptp[1].reshape(8, _FC),
                            ptp[2].reshape(8, _FC)], axis=0)
    sel = _fps_call(p_in).reshape(-1)[:_S]
    pos_y = jnp.take(pos, sel, axis=0)
    batch_y = jnp.take(batch, sel)

    yn = jnp.sum(pos_y ** 2, axis=1)[:, None]
    xn = jnp.sum(pos ** 2, axis=1)[None, :]
    d2 = yn + xn - 2.0 * (pos_y @ pos.T)
    d2 = jnp.where(batch[None, :] == batch_y[:, None], d2, jnp.inf)
    d2p = jnp.pad(d2, ((0, _SPAD - _S), (0, _NPAD - _N)),
                  constant_values=jnp.inf)
    py = jnp.pad(pos_y, ((0, _SPAD - _S), (0, 5)))
    w1p = jnp.pad(W1[128:], ((0, 5), (0, 0)))
    nbr, valid, c = _nbr_call(d2p, py, w1p)

    xp = jnp.pad(x, ((0, _NPAD - _N), (0, 0)))
    pp = jnp.pad(pos, ((0, _NPAD - _N), (0, 5)))
    g1t = _g1_call(xp, pp, W1[:128], w1p)
    idx = nbr.reshape(-1)
    e1 = _sc_gather(g1t, idx)

    b1r = b1.reshape(1, 128)
    ssum1, ssq1, cnt1 = _statsA_call(e1, c, valid, b1r)
    cnt = jnp.maximum(jnp.sum(cnt1[:, 0]), 1.0)
    sc1, sh1 = _bn_coeffs(ssum1, ssq1, cnt, g1, be1)
    z2, ssum2, ssq2 = _passB_call(e1, c, valid, b1r, sc1, sh1, W2,
                                  b2.reshape(1, 128))
    sc2, sh2 = _bn_coeffs(ssum2, ssq2, cnt, g2, be2)
    outp = _passC_call(z2, valid, sc2, sh2, W3, b3.reshape(1, 128))
    return (outp[:_S], pos_y, batch_y)
